# baseline (speedup 1.0000x reference)
import functools

import jax
import jax.numpy as jnp
from jax import lax
from jax.experimental import pallas as pl
from jax.experimental.pallas import tpu as pltpu
from jax.experimental.pallas import tpu_sc as plsc

N_NODES = 10000
N_EDGES = 320000
F_NODE = 128
HID = 16
N_TGT = 128

NC = 2
NS = 16
NW = NC * NS
CH = 512
CHUNKS = 20
EPT = CH * CHUNKS
IDXW = 128
NIR = CH // IDXW
IPT = EPT // IDXW
NIDX = N_EDGES // IDXW
ACC_ROWS = 10112
ACC_PT = ACC_ROWS // NS
G_ROWS = 10240
G_PT = G_ROWS // NS

A8_ROWS = N_EDGES // 8
EPT8 = EPT // 8
CH8 = CH // 8
A8B = 4000
NBLK_A = A8_ROWS // A8B

BN = 2000


def _g_body(x_ref, w_ref, b_ref, o_ref):
    g = (
        jnp.dot(x_ref[...], w_ref[...], preferred_element_type=jnp.float32)
        + b_ref[...]
    )
    o_ref[...] = jnp.concatenate(
        [g, jnp.zeros((G_ROWS - N_NODES, HID), jnp.float32)])


def _a8_body(ea8_ref, wk_ref, o_ref):
    o_ref[...] = jnp.dot(
        ea8_ref[...], wk_ref[...], preferred_element_type=jnp.float32
    )


def _node_body(x_ref, sg_ref, w2_ref, b2_ref, w3a_ref, w3b_ref, b3_ref,
               w4_ref, b4_ref, o_ref):
    sg = sg_ref[...]
    acc = sg[0] + sg[1]
    seg_sum = acc[:, :HID]
    cnt = acc[:, HID:HID + 1]
    mean_r = seg_sum / jnp.maximum(cnt, 1.0)
    m = (
        jnp.dot(mean_r, w2_ref[...], preferred_element_type=jnp.float32)
        + b2_ref[...]
    )
    m = jnp.where(cnt > 0.0, m, 0.0)
    h = (
        jnp.dot(x_ref[...], w3a_ref[...], preferred_element_type=jnp.float32)
        + jnp.dot(m, w3b_ref[...], preferred_element_type=jnp.float32)
        + b3_ref[...]
    )
    h = jnp.maximum(h, 0.0)
    o_ref[...] = (
        jnp.dot(h, w4_ref[...], preferred_element_type=jnp.float32)
        + b4_ref[...]
    )


def _sc_edge_body(g_hbm, at_hbm, ei_hbm, out_hbm,
                  colv, rowv, avt, gv, r32, g_sp, acc_sp,
                  sem_in, sem_g, sem_s):
    c = lax.axis_index("c")
    s = lax.axis_index("s")
    wid = s * NC + c

    zeros16 = jnp.zeros((16,), jnp.float32)
    ones16 = jnp.full((16,), 1.0, jnp.float32)

    def fire_loads(t, b):
        ir = wid * IPT + t * NIR
        abase = wid * EPT8 + t * CH8
        pltpu.async_copy(ei_hbm.at[pl.ds(NIDX + ir, NIR)], colv.at[b],
                         sem_in)
        pltpu.async_copy(ei_hbm.at[pl.ds(ir, NIR)], rowv.at[b], sem_in)
        pltpu.async_copy(at_hbm.at[pl.ds(abase, CH8)], avt.at[b], sem_in)

    def drain_loads(t, b):
        ir = wid * IPT + t * NIR
        abase = wid * EPT8 + t * CH8
        pltpu.make_async_copy(ei_hbm.at[pl.ds(NIDX + ir, NIR)], colv.at[b],
                              sem_in).wait()
        pltpu.make_async_copy(ei_hbm.at[pl.ds(ir, NIR)], rowv.at[b],
                              sem_in).wait()
        pltpu.make_async_copy(at_hbm.at[pl.ds(abase, CH8)], avt.at[b],
                              sem_in).wait()

    def fire_gathers(b):
        for j in range(NIR):
            pltpu.async_copy(g_sp.at[colv.at[b, j]],
                             gv.at[b, pl.ds(j * IDXW, IDXW)], sem_g)

    def drain_gathers(b):
        for j in range(NIR):
            pltpu.make_async_copy(g_sp.at[colv.at[b, j]],
                                  gv.at[b, pl.ds(j * IDXW, IDXW)],
                                  sem_g).wait()

    def fire_scatters(b):
        for j in range(NIR):
            pltpu.async_copy(r32.at[b, pl.ds(j * IDXW, IDXW)],
                             acc_sp.at[rowv.at[b, j]], sem_s, add=True)

    def drain_scatters(b):
        for j in range(NIR):
            pltpu.make_async_copy(r32.at[b, pl.ds(j * IDXW, IDXW)],
                                  acc_sp.at[rowv.at[b, j]], sem_s).wait()

    def compute(b):
        @plsc.parallel_loop(0, CH8, unroll=2)
        def _row(r):
            for k in range(8):
                i = r * 8 + k
                a = avt[b, r, pl.ds(16 * k, 16)]
                r32[b, i, pl.ds(0, 16)] = jnp.maximum(gv[b, i] + a, 0.0)

    @plsc.parallel_loop(0, CH, unroll=8)
    def _zero(i):
        r32[0, i, pl.ds(0, 16)] = zeros16
        r32[0, i, pl.ds(16, 16)] = zeros16
        r32[1, i, pl.ds(0, 16)] = zeros16
        r32[1, i, pl.ds(16, 16)] = zeros16

    fire_loads(0, 0)
    pltpu.sync_copy(r32.at[0, pl.ds(0, ACC_PT)],
                    acc_sp.at[pl.ds(s * ACC_PT, ACC_PT)])
    pltpu.sync_copy(g_hbm.at[pl.ds(s * G_PT, G_PT)],
                    g_sp.at[pl.ds(s * G_PT, G_PT)])

    @plsc.parallel_loop(0, CH, unroll=8)
    def _ones(i):
        r32[0, i, pl.ds(16, 16)] = ones16
        r32[1, i, pl.ds(16, 16)] = ones16

    plsc.subcore_barrier()
    drain_loads(0, 0)
    fire_gathers(0)

    @pl.loop(0, CHUNKS // 2)
    def _pair(tt):
        for sub in range(2):
            b = sub
            t = tt * 2 + sub
            ebase = wid * EPT + t * CH
            vt = ebase < N_EDGES
            vprev = (wid * EPT + (t - 1) * CH) < N_EDGES
            vnext = (wid * EPT + (t + 1) * CH) < N_EDGES

            @pl.when(jnp.logical_and(t >= 1, vprev))
            def _ds():
                drain_scatters(1 - b)

            @pl.when(jnp.logical_and(t + 1 < CHUNKS, vnext))
            def _fl():
                fire_loads(t + 1, 1 - b)

            @pl.when(vt)
            def _main():
                drain_gathers(b)
                compute(b)

            @pl.when(jnp.logical_and(t + 1 < CHUNKS, vnext))
            def _fg():
                drain_loads(t + 1, 1 - b)
                fire_gathers(1 - b)

            @pl.when(vt)
            def _fs():
                fire_scatters(b)

    @pl.when((wid * EPT + (CHUNKS - 1) * CH) < N_EDGES)
    def _final_drain():
        drain_scatters((CHUNKS - 1) % 2)

    plsc.subcore_barrier()
    pltpu.sync_copy(acc_sp.at[pl.ds(s * ACC_PT, ACC_PT)],
                    out_hbm.at[c, pl.ds(s * ACC_PT, ACC_PT)])


@functools.cache
def _make_sc_edge():
    return pl.kernel(
        _sc_edge_body,
        out_type=jax.ShapeDtypeStruct((NC, ACC_ROWS, 32), jnp.float32),
        mesh=plsc.VectorSubcoreMesh(core_axis_name="c", subcore_axis_name="s",
                                    num_cores=NC, num_subcores=NS),
        scratch_types=[
            pltpu.VMEM((2, NIR, IDXW), jnp.int32),
            pltpu.VMEM((2, NIR, IDXW), jnp.int32),
            pltpu.VMEM((2, CH8, 128), jnp.float32),
            pltpu.VMEM((2, CH, HID), jnp.float32),
            pltpu.VMEM((2, CH, 32), jnp.float32),
            pltpu.VMEM_SHARED((G_ROWS, HID), jnp.float32),
            pltpu.VMEM_SHARED((ACC_ROWS, 32), jnp.float32),
            pltpu.SemaphoreType.DMA,
            pltpu.SemaphoreType.DMA,
            pltpu.SemaphoreType.DMA,
        ],
        compiler_params=pltpu.CompilerParams(use_tc_tiling_on_sc=False,
                                             needs_layout_passes=False),
    )


def kernel(x, edge_index, edge_attr, u, batch, W1, b1, W2, b2, W3, b3, W4, b4):
    del u, batch
    W1a, W1b = W1[:F_NODE], W1[F_NODE:]
    W3a, W3b = W3[:F_NODE], W3[F_NODE:]
    ei = edge_index.reshape(2 * NIDX, IDXW)

    G = pl.pallas_call(
        _g_body,
        out_shape=jax.ShapeDtypeStruct((G_ROWS, HID), jnp.float32),
    )(x, W1a, b1.reshape(1, HID))

    EA8 = edge_attr.reshape(A8_ROWS, 128)
    Wk = jnp.kron(jnp.eye(8, dtype=jnp.float32), W1b)
    A8 = pl.pallas_call(
        _a8_body,
        grid=(NBLK_A,),
        in_specs=[
            pl.BlockSpec((A8B, 128), lambda i: (i, 0)),
            pl.BlockSpec((128, 128), lambda i: (0, 0)),
        ],
        out_specs=pl.BlockSpec((A8B, 128), lambda i: (i, 0)),
        out_shape=jax.ShapeDtypeStruct((A8_ROWS, 128), jnp.float32),
    )(EA8, Wk)

    SG = _make_sc_edge()(G, A8, ei)

    out = pl.pallas_call(
        _node_body,
        grid=(N_NODES // BN,),
        in_specs=[
            pl.BlockSpec((BN, F_NODE), lambda i: (i, 0)),
            pl.BlockSpec((NC, BN, 32), lambda i: (0, i, 0)),
            pl.BlockSpec((HID, HID), lambda i: (0, 0)),
            pl.BlockSpec((1, HID), lambda i: (0, 0)),
            pl.BlockSpec((F_NODE, HID), lambda i: (0, 0)),
            pl.BlockSpec((HID, HID), lambda i: (0, 0)),
            pl.BlockSpec((1, HID), lambda i: (0, 0)),
            pl.BlockSpec((HID, N_TGT), lambda i: (0, 0)),
            pl.BlockSpec((1, N_TGT), lambda i: (0, 0)),
        ],
        out_specs=pl.BlockSpec((BN, N_TGT), lambda i: (i, 0)),
        out_shape=jax.ShapeDtypeStruct((N_NODES, N_TGT), jnp.float32),
    )(x, SG, W2, b2.reshape(1, HID), W3a, W3b, b3.reshape(1, HID),
      W4, b4.reshape(1, N_TGT))
    return out


# speedup vs baseline: 1.2182x; 1.2182x over previous
import functools

import jax
import jax.numpy as jnp
from jax import lax
from jax.experimental import pallas as pl
from jax.experimental.pallas import tpu as pltpu
from jax.experimental.pallas import tpu_sc as plsc

N_NODES = 10000
N_EDGES = 320000
F_NODE = 128
HID = 16
N_TGT = 128

NC = 2
NS = 16
NW = NC * NS
CH = 512
CHUNKS = 20
EPT = CH * CHUNKS
IDXW = 128
NIR = CH // IDXW
IPT = EPT // IDXW
NIDX = N_EDGES // IDXW
ACC_ROWS = 10112
ACC_PT = ACC_ROWS // NS
G_ROWS = 10240
G_PT = G_ROWS // NS

ATB = 32000
NBLK_A = N_EDGES // ATB

BN = 2000


def _g_body(x_ref, w_ref, b_ref, o_ref):
    g = (
        jnp.dot(x_ref[...], w_ref[...], preferred_element_type=jnp.float32)
        + b_ref[...]
    )
    o_ref[...] = jnp.concatenate(
        [g, jnp.zeros((G_ROWS - N_NODES, HID), jnp.float32)])


def _at_body(w_ref, eat_ref, o_ref):
    o_ref[...] = jnp.dot(
        w_ref[...], eat_ref[...], preferred_element_type=jnp.float32
    )


def _node_body(x_ref, sg_ref, w2_ref, b2_ref, w3a_ref, w3b_ref, b3_ref,
               w4_ref, b4_ref, o_ref):
    sg = sg_ref[...]
    acc = sg[0] + sg[1]
    seg_sum = acc[:, :HID]
    cnt = acc[:, HID:HID + 1]
    mean_r = seg_sum / jnp.maximum(cnt, 1.0)
    m = (
        jnp.dot(mean_r, w2_ref[...], preferred_element_type=jnp.float32)
        + b2_ref[...]
    )
    m = jnp.where(cnt > 0.0, m, 0.0)
    h = (
        jnp.dot(x_ref[...], w3a_ref[...], preferred_element_type=jnp.float32)
        + jnp.dot(m, w3b_ref[...], preferred_element_type=jnp.float32)
        + b3_ref[...]
    )
    h = jnp.maximum(h, 0.0)
    o_ref[...] = (
        jnp.dot(h, w4_ref[...], preferred_element_type=jnp.float32)
        + b4_ref[...]
    )


def _sc_edge_body(g_hbm, at_hbm, ei_hbm, out_hbm,
                  colv, rowv, avt, gv, r32, g_sp, acc_sp,
                  sem_in, sem_g, sem_s):
    c = lax.axis_index("c")
    s = lax.axis_index("s")
    wid = s * NC + c

    zeros16 = jnp.zeros((16,), jnp.float32)
    ones16 = jnp.full((16,), 1.0, jnp.float32)
    lanes = jnp.arange(16, dtype=jnp.int32)

    def fire_loads(t, b):
        ir = wid * IPT + t * NIR
        ebase = wid * EPT + t * CH
        pltpu.async_copy(ei_hbm.at[pl.ds(NIDX + ir, NIR)], colv.at[b],
                         sem_in)
        pltpu.async_copy(ei_hbm.at[pl.ds(ir, NIR)], rowv.at[b], sem_in)
        pltpu.async_copy(at_hbm.at[:, pl.ds(ebase, CH)], avt.at[b], sem_in)

    def drain_loads(t, b):
        ir = wid * IPT + t * NIR
        ebase = wid * EPT + t * CH
        pltpu.make_async_copy(ei_hbm.at[pl.ds(NIDX + ir, NIR)], colv.at[b],
                              sem_in).wait()
        pltpu.make_async_copy(ei_hbm.at[pl.ds(ir, NIR)], rowv.at[b],
                              sem_in).wait()
        pltpu.make_async_copy(at_hbm.at[:, pl.ds(ebase, CH)], avt.at[b],
                              sem_in).wait()

    def fire_gathers(b):
        for j in range(NIR):
            pltpu.async_copy(g_sp.at[colv.at[b, j]],
                             gv.at[b, pl.ds(j * IDXW, IDXW)], sem_g)

    def drain_gathers(b):
        for j in range(NIR):
            pltpu.make_async_copy(g_sp.at[colv.at[b, j]],
                                  gv.at[b, pl.ds(j * IDXW, IDXW)],
                                  sem_g).wait()

    def fire_scatters(b):
        for j in range(NIR):
            pltpu.async_copy(r32.at[b, pl.ds(j * IDXW, IDXW)],
                             acc_sp.at[rowv.at[b, j]], sem_s, add=True)

    def drain_scatters(b):
        for j in range(NIR):
            pltpu.make_async_copy(r32.at[b, pl.ds(j * IDXW, IDXW)],
                                  acc_sp.at[rowv.at[b, j]], sem_s).wait()

    def compute(b):
        @plsc.parallel_loop(0, CH, unroll=8)
        def _edge(i):
            a = plsc.load_gather(avt.at[b],
                                 [lanes, jnp.full((16,), i, jnp.int32)])
            r32[b, i, pl.ds(0, 16)] = jnp.maximum(gv[b, i] + a, 0.0)

    @plsc.parallel_loop(0, CH, unroll=8)
    def _zero(i):
        r32[0, i, pl.ds(0, 16)] = zeros16
        r32[0, i, pl.ds(16, 16)] = zeros16
        r32[1, i, pl.ds(0, 16)] = zeros16
        r32[1, i, pl.ds(16, 16)] = zeros16

    fire_loads(0, 0)
    pltpu.sync_copy(r32.at[0, pl.ds(0, ACC_PT)],
                    acc_sp.at[pl.ds(s * ACC_PT, ACC_PT)])
    pltpu.sync_copy(g_hbm.at[pl.ds(s * G_PT, G_PT)],
                    g_sp.at[pl.ds(s * G_PT, G_PT)])

    @plsc.parallel_loop(0, CH, unroll=8)
    def _ones(i):
        r32[0, i, pl.ds(16, 16)] = ones16
        r32[1, i, pl.ds(16, 16)] = ones16

    plsc.subcore_barrier()
    drain_loads(0, 0)
    fire_gathers(0)

    @pl.loop(0, CHUNKS // 2)
    def _pair(tt):
        for sub in range(2):
            b = sub
            t = tt * 2 + sub
            ebase = wid * EPT + t * CH
            vt = ebase < N_EDGES
            vprev = (wid * EPT + (t - 1) * CH) < N_EDGES
            vnext = (wid * EPT + (t + 1) * CH) < N_EDGES

            @pl.when(jnp.logical_and(t >= 1, vprev))
            def _ds():
                drain_scatters(1 - b)

            @pl.when(jnp.logical_and(t + 1 < CHUNKS, vnext))
            def _fl():
                fire_loads(t + 1, 1 - b)

            @pl.when(vt)
            def _main():
                drain_gathers(b)
                compute(b)

            @pl.when(jnp.logical_and(t + 1 < CHUNKS, vnext))
            def _fg():
                drain_loads(t + 1, 1 - b)
                fire_gathers(1 - b)

            @pl.when(vt)
            def _fs():
                fire_scatters(b)

    @pl.when((wid * EPT + (CHUNKS - 1) * CH) < N_EDGES)
    def _final_drain():
        drain_scatters((CHUNKS - 1) % 2)

    plsc.subcore_barrier()
    pltpu.sync_copy(acc_sp.at[pl.ds(s * ACC_PT, ACC_PT)],
                    out_hbm.at[c, pl.ds(s * ACC_PT, ACC_PT)])


@functools.cache
def _make_sc_edge():
    return pl.kernel(
        _sc_edge_body,
        out_type=jax.ShapeDtypeStruct((NC, ACC_ROWS, 32), jnp.float32),
        mesh=plsc.VectorSubcoreMesh(core_axis_name="c", subcore_axis_name="s",
                                    num_cores=NC, num_subcores=NS),
        scratch_types=[
            pltpu.VMEM((2, NIR, IDXW), jnp.int32),
            pltpu.VMEM((2, NIR, IDXW), jnp.int32),
            pltpu.VMEM((2, HID, CH), jnp.float32),
            pltpu.VMEM((2, CH, HID), jnp.float32),
            pltpu.VMEM((2, CH, 32), jnp.float32),
            pltpu.VMEM_SHARED((G_ROWS, HID), jnp.float32),
            pltpu.VMEM_SHARED((ACC_ROWS, 32), jnp.float32),
            pltpu.SemaphoreType.DMA,
            pltpu.SemaphoreType.DMA,
            pltpu.SemaphoreType.DMA,
        ],
        compiler_params=pltpu.CompilerParams(use_tc_tiling_on_sc=False,
                                             needs_layout_passes=False),
    )


def kernel(x, edge_index, edge_attr, u, batch, W1, b1, W2, b2, W3, b3, W4, b4):
    del u, batch
    W1a, W1b = W1[:F_NODE], W1[F_NODE:]
    W3a, W3b = W3[:F_NODE], W3[F_NODE:]
    ei = edge_index.reshape(2 * NIDX, IDXW)

    G = pl.pallas_call(
        _g_body,
        out_shape=jax.ShapeDtypeStruct((G_ROWS, HID), jnp.float32),
    )(x, W1a, b1.reshape(1, HID))

    A_t = pl.pallas_call(
        _at_body,
        grid=(NBLK_A,),
        in_specs=[
            pl.BlockSpec((HID, HID), lambda i: (0, 0)),
            pl.BlockSpec((HID, ATB), lambda i: (0, i)),
        ],
        out_specs=pl.BlockSpec((HID, ATB), lambda i: (0, i)),
        out_shape=jax.ShapeDtypeStruct((HID, N_EDGES), jnp.float32),
    )(W1b.T, edge_attr.T)

    SG = _make_sc_edge()(G, A_t, ei)

    out = pl.pallas_call(
        _node_body,
        grid=(N_NODES // BN,),
        in_specs=[
            pl.BlockSpec((BN, F_NODE), lambda i: (i, 0)),
            pl.BlockSpec((NC, BN, 32), lambda i: (0, i, 0)),
            pl.BlockSpec((HID, HID), lambda i: (0, 0)),
            pl.BlockSpec((1, HID), lambda i: (0, 0)),
            pl.BlockSpec((F_NODE, HID), lambda i: (0, 0)),
            pl.BlockSpec((HID, HID), lambda i: (0, 0)),
            pl.BlockSpec((1, HID), lambda i: (0, 0)),
            pl.BlockSpec((HID, N_TGT), lambda i: (0, 0)),
            pl.BlockSpec((1, N_TGT), lambda i: (0, 0)),
        ],
        out_specs=pl.BlockSpec((BN, N_TGT), lambda i: (i, 0)),
        out_shape=jax.ShapeDtypeStruct((N_NODES, N_TGT), jnp.float32),
    )(x, SG, W2, b2.reshape(1, HID), W3a, W3b, b3.reshape(1, HID),
      W4, b4.reshape(1, N_TGT))
    return out
